# baseline (device time: 105130 ns/iter reference)
import jax
import jax.numpy as jnp
from jax import lax
from jax.experimental import pallas as pl
from jax.experimental.pallas import tpu as pltpu

N_DEV = 16
HOPS = 8
SUBS = 2
R8_SUBS = tuple(range(SUBS // 2))
L8_SUBS = tuple(range(SUBS // 2, SUBS))
DOT_DIMS = (((1,), (0,)), ((), ()))


def kernel(x, w_mat, scale_x, scale_w):
    m_per, k = x.shape
    _, n_per = w_mat.shape

    x8 = x.astype(jnp.float8_e4m3fn)
    w8 = w_mat.astype(jnp.float8_e5m2)

    def body(x_ref, w_ref, sx_ref, sw_ref, out_ref,
             agr_ref, agl_ref, sendr_sems, recvr_sems, sendl_sems, recvl_sems):
        my = lax.axis_index("i")
        left = lax.rem(my + N_DEV - 1, N_DEV)
        right = lax.rem(my + 1, N_DEV)

        barrier_sem = pltpu.get_barrier_semaphore()
        for nbr in (left, right):
            pl.semaphore_signal(
                barrier_sem, inc=1,
                device_id=(nbr,), device_id_type=pl.DeviceIdType.MESH,
            )
        pl.semaphore_wait(barrier_sem, 2)

        scale = sx_ref[0] * sw_ref[0]
        w_val = w_ref[...]

        m_sub = m_per // SUBS

        agr_ref[0, :, :] = x_ref[...]
        agl_ref[0, :, :] = x_ref[...]

        def make_r(h, i):
            rows = pl.ds(i * m_sub, m_sub)
            return pltpu.make_async_remote_copy(
                src_ref=agr_ref.at[h - 1, rows],
                dst_ref=agr_ref.at[h, rows],
                send_sem=sendr_sems.at[h - 1, i],
                recv_sem=recvr_sems.at[h - 1, i],
                device_id=(right,),
                device_id_type=pl.DeviceIdType.MESH,
            )

        def make_l(h, i):
            rows = pl.ds(i * m_sub, m_sub)
            return pltpu.make_async_remote_copy(
                src_ref=agl_ref.at[h - 1, rows],
                dst_ref=agl_ref.at[h, rows],
                send_sem=sendl_sems.at[h - 1, i],
                recv_sem=recvl_sems.at[h - 1, i],
                device_id=(left,),
                device_id_type=pl.DeviceIdType.MESH,
            )

        rdmas_r = [
            [make_r(h, i) for i in (range(SUBS) if h < HOPS else R8_SUBS)]
            for h in range(1, HOPS + 1)
        ]
        rdmas_l = [
            [make_l(h, i) for i in (range(SUBS) if h < HOPS else L8_SUBS)]
            for h in range(1, HOPS + 1)
        ]

        for i in range(SUBS):
            rdmas_r[0][i].start()
            rdmas_l[0][i].start()

        acc = lax.dot_general(
            x_ref[...], w_val, DOT_DIMS, preferred_element_type=jnp.float32
        )
        out_ref[pl.ds(my * m_per, m_per), :] = jnp.maximum(acc * scale, 0.0)

        for h in range(1, HOPS):
            for i in range(SUBS):
                rdmas_r[h - 1][i].wait_recv()
                if h + 1 < HOPS or i in R8_SUBS:
                    nxt = i if h + 1 < HOPS else R8_SUBS.index(i)
                    rdmas_r[h][nxt].start()
                rdmas_l[h - 1][i].wait_recv()
                if h + 1 < HOPS or i in L8_SUBS:
                    nxt = i if h + 1 < HOPS else L8_SUBS.index(i)
                    rdmas_l[h][nxt].start()

            origin_r = lax.rem(my + N_DEV - h, N_DEV)
            acc = lax.dot_general(
                agr_ref[h], w_val, DOT_DIMS, preferred_element_type=jnp.float32
            )
            out_ref[pl.ds(origin_r * m_per, m_per), :] = jnp.maximum(acc * scale, 0.0)

            origin_l = lax.rem(my + h, N_DEV)
            acc = lax.dot_general(
                agl_ref[h], w_val, DOT_DIMS, preferred_element_type=jnp.float32
            )
            out_ref[pl.ds(origin_l * m_per, m_per), :] = jnp.maximum(acc * scale, 0.0)

        for rdma in rdmas_r[HOPS - 1] + rdmas_l[HOPS - 1]:
            rdma.wait_recv()
        m_half = m_per // 2
        origin8 = lax.rem(my + HOPS, N_DEV)
        acc = lax.dot_general(
            agr_ref[HOPS, :m_half], w_val, DOT_DIMS,
            preferred_element_type=jnp.float32,
        )
        out_ref[pl.ds(origin8 * m_per, m_half), :] = jnp.maximum(acc * scale, 0.0)
        acc = lax.dot_general(
            agl_ref[HOPS, m_half:], w_val, DOT_DIMS,
            preferred_element_type=jnp.float32,
        )
        out_ref[pl.ds(origin8 * m_per + m_half, m_half), :] = jnp.maximum(
            acc * scale, 0.0
        )

        for hop in rdmas_r + rdmas_l:
            for rdma in hop:
                rdma.wait_send()

    return pl.pallas_call(
        body,
        out_shape=jax.ShapeDtypeStruct((N_DEV * m_per, n_per), jnp.float32),
        in_specs=[
            pl.BlockSpec(memory_space=pltpu.VMEM),
            pl.BlockSpec(memory_space=pltpu.VMEM),
            pl.BlockSpec(memory_space=pltpu.SMEM),
            pl.BlockSpec(memory_space=pltpu.SMEM),
        ],
        out_specs=pl.BlockSpec(memory_space=pltpu.VMEM),
        scratch_shapes=[
            pltpu.VMEM((HOPS + 1, m_per, k), jnp.float8_e4m3fn),
            pltpu.VMEM((HOPS + 1, m_per, k), jnp.float8_e4m3fn),
            pltpu.SemaphoreType.DMA((HOPS, SUBS)),
            pltpu.SemaphoreType.DMA((HOPS, SUBS)),
            pltpu.SemaphoreType.DMA((HOPS, SUBS)),
            pltpu.SemaphoreType.DMA((HOPS, SUBS)),
        ],
        compiler_params=pltpu.CompilerParams(collective_id=0),
    )(x8, w8, scale_x, scale_w)


# device time: 101308 ns/iter; 1.0377x vs baseline; 1.0377x over previous
import jax
import jax.numpy as jnp
from jax import lax
from jax.experimental import pallas as pl
from jax.experimental.pallas import tpu as pltpu

N_DEV = 16
HOPS = 8
SUBS = 2
R8_SUBS = tuple(range(SUBS // 2))
L8_SUBS = tuple(range(SUBS // 2, SUBS))
DOT_DIMS = (((1,), (0,)), ((), ()))


def kernel(x, w_mat, scale_x, scale_w):
    m_per, k = x.shape
    _, n_per = w_mat.shape

    x8 = x.astype(jnp.float8_e4m3fn)

    def body(x_ref, w_ref, sx_ref, sw_ref, out_ref,
             agr_ref, agl_ref, w32_ref, w_sem,
             sendr_sems, recvr_sems, sendl_sems, recvl_sems):
        my = lax.axis_index("i")
        left = lax.rem(my + N_DEV - 1, N_DEV)
        right = lax.rem(my + 1, N_DEV)

        barrier_sem = pltpu.get_barrier_semaphore()
        for nbr in (left, right):
            pl.semaphore_signal(
                barrier_sem, inc=1,
                device_id=(nbr,), device_id_type=pl.DeviceIdType.MESH,
            )
        pl.semaphore_wait(barrier_sem, 2)

        m_sub = m_per // SUBS

        def make_r(h, i):
            rows = pl.ds(i * m_sub, m_sub)
            return pltpu.make_async_remote_copy(
                src_ref=x_ref.at[rows] if h == 1 else agr_ref.at[h - 1, rows],
                dst_ref=agr_ref.at[h, rows],
                send_sem=sendr_sems.at[h - 1, i],
                recv_sem=recvr_sems.at[h - 1, i],
                device_id=(right,),
                device_id_type=pl.DeviceIdType.MESH,
            )

        def make_l(h, i):
            rows = pl.ds(i * m_sub, m_sub)
            return pltpu.make_async_remote_copy(
                src_ref=x_ref.at[rows] if h == 1 else agl_ref.at[h - 1, rows],
                dst_ref=agl_ref.at[h, rows],
                send_sem=sendl_sems.at[h - 1, i],
                recv_sem=recvl_sems.at[h - 1, i],
                device_id=(left,),
                device_id_type=pl.DeviceIdType.MESH,
            )

        rdmas_r = [
            [make_r(h, i) for i in (range(SUBS) if h < HOPS else R8_SUBS)]
            for h in range(1, HOPS + 1)
        ]
        rdmas_l = [
            [make_l(h, i) for i in (range(SUBS) if h < HOPS else L8_SUBS)]
            for h in range(1, HOPS + 1)
        ]

        for i in range(SUBS):
            rdmas_r[0][i].start()
            rdmas_l[0][i].start()

        w_dma = pltpu.make_async_copy(w_ref, w32_ref, w_sem)
        w_dma.start()
        scale = sx_ref[0] * sw_ref[0]
        w_dma.wait()
        w_val = w32_ref[...].astype(jnp.float8_e5m2)

        acc = lax.dot_general(
            x_ref[...], w_val, DOT_DIMS, preferred_element_type=jnp.float32
        )
        out_ref[pl.ds(my * m_per, m_per), :] = jnp.maximum(acc * scale, 0.0)

        for h in range(1, HOPS):
            for i in range(SUBS):
                rdmas_r[h - 1][i].wait_recv()
                if h + 1 < HOPS or i in R8_SUBS:
                    nxt = i if h + 1 < HOPS else R8_SUBS.index(i)
                    rdmas_r[h][nxt].start()
                rdmas_l[h - 1][i].wait_recv()
                if h + 1 < HOPS or i in L8_SUBS:
                    nxt = i if h + 1 < HOPS else L8_SUBS.index(i)
                    rdmas_l[h][nxt].start()

            origin_r = lax.rem(my + N_DEV - h, N_DEV)
            acc = lax.dot_general(
                agr_ref[h], w_val, DOT_DIMS, preferred_element_type=jnp.float32
            )
            out_ref[pl.ds(origin_r * m_per, m_per), :] = jnp.maximum(acc * scale, 0.0)

            origin_l = lax.rem(my + h, N_DEV)
            acc = lax.dot_general(
                agl_ref[h], w_val, DOT_DIMS, preferred_element_type=jnp.float32
            )
            out_ref[pl.ds(origin_l * m_per, m_per), :] = jnp.maximum(acc * scale, 0.0)

        for rdma in rdmas_r[HOPS - 1] + rdmas_l[HOPS - 1]:
            rdma.wait_recv()
        m_half = m_per // 2
        origin8 = lax.rem(my + HOPS, N_DEV)
        acc = lax.dot_general(
            agr_ref[HOPS, :m_half], w_val, DOT_DIMS,
            preferred_element_type=jnp.float32,
        )
        out_ref[pl.ds(origin8 * m_per, m_half), :] = jnp.maximum(acc * scale, 0.0)
        acc = lax.dot_general(
            agl_ref[HOPS, m_half:], w_val, DOT_DIMS,
            preferred_element_type=jnp.float32,
        )
        out_ref[pl.ds(origin8 * m_per + m_half, m_half), :] = jnp.maximum(
            acc * scale, 0.0
        )

        for hop in rdmas_r + rdmas_l:
            for rdma in hop:
                rdma.wait_send()

    return pl.pallas_call(
        body,
        out_shape=jax.ShapeDtypeStruct((N_DEV * m_per, n_per), jnp.float32),
        in_specs=[
            pl.BlockSpec(memory_space=pltpu.VMEM),
            pl.BlockSpec(memory_space=pl.ANY),
            pl.BlockSpec(memory_space=pltpu.SMEM),
            pl.BlockSpec(memory_space=pltpu.SMEM),
        ],
        out_specs=pl.BlockSpec(memory_space=pltpu.VMEM),
        scratch_shapes=[
            pltpu.VMEM((HOPS + 1, m_per, k), jnp.float8_e4m3fn),
            pltpu.VMEM((HOPS + 1, m_per, k), jnp.float8_e4m3fn),
            pltpu.VMEM((k, n_per), jnp.float32),
            pltpu.SemaphoreType.DMA,
            pltpu.SemaphoreType.DMA((HOPS, SUBS)),
            pltpu.SemaphoreType.DMA((HOPS, SUBS)),
            pltpu.SemaphoreType.DMA((HOPS, SUBS)),
            pltpu.SemaphoreType.DMA((HOPS, SUBS)),
        ],
        compiler_params=pltpu.CompilerParams(collective_id=0),
    )(x8, w_mat, scale_x, scale_w)
